# baseline (device time: 26255 ns/iter reference)
import jax
import jax.numpy as jnp
from jax import lax
from jax.experimental import pallas as pl
from jax.experimental.pallas import tpu as pltpu

N_DEV = 4
M_CHUNK = 512
MH = M_CHUNK // 2
D = 512
EPS = 1e-6


def kernel(partial, gamma):
    gamma2d = gamma.reshape(1, D)

    def body(x_hbm, g_hbm, out_hbm, lx, gv, rp1, rp2, stage, oy,
             ld_sems, p1_send, p1_recv, p2_send, p2_recv):
        my = lax.axis_index("i")
        pa = my ^ 1
        pb = 3 - my

        def sub_at(c, s):
            return x_hbm.at[0, pl.ds(c * M_CHUNK + s * MH, MH), :]

        ld0 = pltpu.make_async_copy(sub_at(3 - my, 0), lx.at[0], ld_sems.at[0])
        ld1 = pltpu.make_async_copy(sub_at(my ^ 1, 1), lx.at[1], ld_sems.at[1])
        ld2 = pltpu.make_async_copy(sub_at(my, 0), lx.at[2], ld_sems.at[2])
        ld3 = pltpu.make_async_copy(sub_at(my, 1), lx.at[3], ld_sems.at[3])
        ldg = pltpu.make_async_copy(g_hbm, gv, ld_sems.at[4])
        for ld in (ld0, ld1, ld2, ld3, ldg):
            ld.start()

        barrier_sem = pltpu.get_barrier_semaphore()
        for nbr in (pa, pb):
            pl.semaphore_signal(
                barrier_sem, inc=1,
                device_id=(nbr,), device_id_type=pl.DeviceIdType.MESH,
            )
        pl.semaphore_wait(barrier_sem, 2)

        def p1_rdma(slot, c, s, target):
            return pltpu.make_async_remote_copy(
                src_ref=sub_at(c, s),
                dst_ref=rp1.at[slot],
                send_sem=p1_send.at[slot],
                recv_sem=p1_recv.at[slot],
                device_id=(target,),
                device_id_type=pl.DeviceIdType.MESH,
            )

        rB = p1_rdma(0, 3 - pa, 0, pa)
        rD = p1_rdma(2, pb ^ 1, 1, pb)
        rA = p1_rdma(1, pa, 0, pa)
        rC = p1_rdma(3, pb, 1, pb)
        rB.start()
        rD.start()
        rA.start()
        rC.start()

        def p2_rdma(slot, target):
            return pltpu.make_async_remote_copy(
                src_ref=stage.at[slot],
                dst_ref=rp2.at[slot],
                send_sem=p2_send.at[slot],
                recv_sem=p2_recv.at[slot],
                device_id=(target,),
                device_id_type=pl.DeviceIdType.MESH,
            )

        rB.wait_recv()
        ld0.wait()
        stage[0] = lx[0] + rp1[0]
        r3 = p2_rdma(0, pb)
        r3.start()

        rD.wait_recv()
        ld1.wait()
        stage[1] = lx[1] + rp1[2]
        r4 = p2_rdma(1, pa)
        r4.start()

        rA.wait_recv()
        ld2.wait()
        a0 = lx[2] + rp1[1]
        rC.wait_recv()
        ld3.wait()
        a1 = lx[3] + rp1[3]
        ldg.wait()

        def norm(y):
            ms = jnp.mean(y * y, axis=-1, keepdims=True)
            return y * lax.rsqrt(ms + EPS) * gv[...]

        r3.wait_recv()
        oy[0] = norm(a0 + rp2[0])
        st0 = pltpu.make_async_copy(
            oy.at[0], out_hbm.at[0:MH, :], ld_sems.at[5])
        st0.start()
        r4.wait_recv()
        oy[1] = norm(a1 + rp2[1])
        st1 = pltpu.make_async_copy(
            oy.at[1], out_hbm.at[MH:M_CHUNK, :], ld_sems.at[6])
        st1.start()
        st0.wait()
        st1.wait()

        for r in (rB, rD, rA, rC, r3, r4):
            r.wait_send()

    return pl.pallas_call(
        body,
        out_shape=jax.ShapeDtypeStruct((M_CHUNK, D), jnp.float32),
        in_specs=[
            pl.BlockSpec(memory_space=pl.ANY),
            pl.BlockSpec(memory_space=pl.ANY),
        ],
        out_specs=pl.BlockSpec(memory_space=pl.ANY),
        scratch_shapes=[
            pltpu.VMEM((4, MH, D), jnp.float32),
            pltpu.VMEM((1, D), jnp.float32),
            pltpu.VMEM((4, MH, D), jnp.float32),
            pltpu.VMEM((2, MH, D), jnp.float32),
            pltpu.VMEM((2, MH, D), jnp.float32),
            pltpu.VMEM((2, MH, D), jnp.float32),
            pltpu.SemaphoreType.DMA((7,)),
            pltpu.SemaphoreType.DMA((4,)),
            pltpu.SemaphoreType.DMA((4,)),
            pltpu.SemaphoreType.DMA((2,)),
            pltpu.SemaphoreType.DMA((2,)),
        ],
        compiler_params=pltpu.CompilerParams(collective_id=0),
    )(partial, gamma2d)


# device time: 25885 ns/iter; 1.0143x vs baseline; 1.0143x over previous
import jax
import jax.numpy as jnp
from jax import lax
from jax.experimental import pallas as pl
from jax.experimental.pallas import tpu as pltpu

N_DEV = 4
M_CHUNK = 512
MH = M_CHUNK // 2
D = 512
EPS = 1e-6
SEGS = ((0, 192), (192, 64))


def kernel(partial, gamma):
    gamma2d = gamma.reshape(1, D)

    def body(x_ref, g_ref, out_ref, rp1, rp2, stage,
             p1_send, p1_recv, p2_send, p2_recv):
        my = lax.axis_index("i")
        pa = my ^ 1
        pb = 3 - my

        barrier_sem = pltpu.get_barrier_semaphore()
        for nbr in (pa, pb):
            pl.semaphore_signal(
                barrier_sem, inc=1,
                device_id=(nbr,), device_id_type=pl.DeviceIdType.MESH,
            )
        pl.semaphore_wait(barrier_sem, 2)

        def sub_at(c, s):
            return x_ref.at[0, pl.ds(c * M_CHUNK + s * MH, MH), :]

        def sub(c, s):
            return x_ref[0, pl.ds(c * M_CHUNK + s * MH, MH), :]

        def p1_rdma(slot, c, s, target):
            return pltpu.make_async_remote_copy(
                src_ref=sub_at(c, s),
                dst_ref=rp1.at[slot],
                send_sem=p1_send.at[slot],
                recv_sem=p1_recv.at[slot],
                device_id=(target,),
                device_id_type=pl.DeviceIdType.MESH,
            )

        rB = p1_rdma(0, 3 - pa, 0, pa)
        rD = p1_rdma(2, pb ^ 1, 1, pb)
        rA = p1_rdma(1, pa, 0, pa)
        rC = p1_rdma(3, pb, 1, pb)
        rB.start()
        rD.start()
        rA.start()
        rC.start()

        def p2_rdma(stream, seg, target):
            lo, n = SEGS[seg]
            return pltpu.make_async_remote_copy(
                src_ref=stage.at[stream, pl.ds(lo, n)],
                dst_ref=rp2.at[stream, pl.ds(lo, n)],
                send_sem=p2_send.at[stream, seg],
                recv_sem=p2_recv.at[stream, seg],
                device_id=(target,),
                device_id_type=pl.DeviceIdType.MESH,
            )

        rB.wait_recv()
        stage[0] = sub(3 - my, 0) + rp1[0]
        r3a = p2_rdma(0, 0, pb)
        r3b = p2_rdma(0, 1, pb)
        r3a.start()
        r3b.start()

        rD.wait_recv()
        stage[1] = sub(my ^ 1, 1) + rp1[2]
        r4a = p2_rdma(1, 0, pa)
        r4b = p2_rdma(1, 1, pa)
        r4a.start()
        r4b.start()

        rA.wait_recv()
        a0 = sub(my, 0) + rp1[1]
        rC.wait_recv()
        a1 = sub(my, 1) + rp1[3]

        def norm(y):
            ms = jnp.mean(y * y, axis=-1, keepdims=True)
            return y * lax.rsqrt(ms + EPS) * g_ref[...]

        lo0, n0 = SEGS[0]
        lo1, n1 = SEGS[1]
        r3a.wait_recv()
        out_ref[lo0:lo0 + n0, :] = norm(a0[lo0:lo0 + n0] + rp2[0, lo0:lo0 + n0])
        r4a.wait_recv()
        out_ref[MH + lo0:MH + lo0 + n0, :] = norm(
            a1[lo0:lo0 + n0] + rp2[1, lo0:lo0 + n0])
        r3b.wait_recv()
        out_ref[lo1:lo1 + n1, :] = norm(a0[lo1:lo1 + n1] + rp2[0, lo1:lo1 + n1])
        r4b.wait_recv()
        out_ref[MH + lo1:MH + lo1 + n1, :] = norm(
            a1[lo1:lo1 + n1] + rp2[1, lo1:lo1 + n1])

        for r in (rB, rD, rA, rC, r3a, r3b, r4a, r4b):
            r.wait_send()

    return pl.pallas_call(
        body,
        out_shape=jax.ShapeDtypeStruct((M_CHUNK, D), jnp.float32),
        in_specs=[
            pl.BlockSpec(memory_space=pltpu.VMEM),
            pl.BlockSpec(memory_space=pltpu.VMEM),
        ],
        out_specs=pl.BlockSpec(memory_space=pltpu.VMEM),
        scratch_shapes=[
            pltpu.VMEM((4, MH, D), jnp.float32),
            pltpu.VMEM((2, MH, D), jnp.float32),
            pltpu.VMEM((2, MH, D), jnp.float32),
            pltpu.SemaphoreType.DMA((4,)),
            pltpu.SemaphoreType.DMA((4,)),
            pltpu.SemaphoreType.DMA((2, 2)),
            pltpu.SemaphoreType.DMA((2, 2)),
        ],
        compiler_params=pltpu.CompilerParams(collective_id=0),
    )(partial, gamma2d)
